# baseline (speedup 1.0000x reference)
import jax
import jax.numpy as jnp
from jax import lax
from jax.experimental import pallas as pl
from jax.experimental.pallas import tpu as pltpu
from jax.experimental.pallas import tpu_sc as plsc

B = 30000
S = 16
F = 128
E = 64
NC = 2
NS = 16
NW = NC * NS
BPW = 960
NFULL = B // BPW
BT = B - NFULL * BPW
G_MAIN = 96
G_TAIL = 48
NACC = 4
NSB = 2


def _emit_pipeline(feat_hbm, self_out, sum_out, idxn_v, nodes_v,
                   accs, sbufs, nsem, wsem, ssem, swsem, base, g, nchunk):
  pend_init = {}
  pend_write = {}
  pend_sg = {}
  pend_sw = {}
  waited_write = set()
  waited_sw = set()

  def acc_ref(sl):
    return accs[sl].at[pl.ds(0, g)] if g != accs[sl].shape[0] else accs[sl]

  def sbuf_ref(sl):
    return sbufs[sl].at[pl.ds(0, g)] if g != sbufs[sl].shape[0] else sbufs[sl]

  for c in range(min(NACC, nchunk)):
    pend_init[c] = pltpu.async_copy(
        feat_hbm.at[idxn_v.at[pl.ds(c * g, g)]], acc_ref(c % NACC),
        nsem[c % NACC])
  for c in range(min(NSB, nchunk)):
    pend_sg[c] = pltpu.async_copy(
        feat_hbm.at[nodes_v.at[pl.ds(c * g, g)]], sbuf_ref(c % NSB),
        ssem[c % NSB])

  for c in range(nchunk):
    sl = c % NACC
    ssl = c % NSB
    pend_init[c].wait()
    adds = [
        pltpu.async_copy(feat_hbm.at[idxn_v.at[pl.ds(j * BPW + c * g, g)]],
                         acc_ref(sl), nsem[sl], add=True)
        for j in range(1, S)
    ]
    pend_sg[c].wait()
    pend_sw[c] = pltpu.async_copy(
        sbuf_ref(ssl), self_out.at[pl.ds(base + c * g, g)], swsem[ssl])
    if c + NSB < nchunk:
      pend_sw[c].wait()
      waited_sw.add(c)
      pend_sg[c + NSB] = pltpu.async_copy(
          feat_hbm.at[nodes_v.at[pl.ds((c + NSB) * g, g)]],
          sbuf_ref((c + NSB) % NSB), ssem[(c + NSB) % NSB])
    nxt = c + NACC - 1
    if c >= 1 and nxt < nchunk:
      pend_write[c - 1].wait()
      waited_write.add(c - 1)
      pend_init[nxt] = pltpu.async_copy(
          feat_hbm.at[idxn_v.at[pl.ds(nxt * g, g)]], acc_ref(nxt % NACC),
          nsem[nxt % NACC])
    for a in adds:
      a.wait()
    pend_write[c] = pltpu.async_copy(
        acc_ref(sl), sum_out.at[pl.ds(base + c * g, g)], wsem[sl])

  for c in range(nchunk):
    if c in pend_write and c not in waited_write:
      pend_write[c].wait()
    if c in pend_sw and c not in waited_sw:
      pend_sw[c].wait()


def _sc_body(nodes_hbm, neigh_hbm, feat_hbm, self_out, sum_out,
             idxn_v, nodes_v, acc0, acc1, acc2, acc3, sb0, sb1,
             nsem0, nsem1, nsem2, nsem3, wsem0, wsem1, wsem2, wsem3,
             ssem0, ssem1, swsem0, swsem1):
  accs = [acc0, acc1, acc2, acc3]
  nsem = [nsem0, nsem1, nsem2, nsem3]
  wsem = [wsem0, wsem1, wsem2, wsem3]
  sbufs = [sb0, sb1]
  ssem = [ssem0, ssem1]
  swsem = [swsem0, swsem1]

  c_ax = lax.axis_index("c")
  s_ax = lax.axis_index("s")
  wid = s_ax * NC + c_ax
  @pl.when(wid < NFULL)
  def _full():
    base = wid * BPW
    idx_loads = [
        pltpu.async_copy(neigh_hbm.at[pl.ds(j * B + base, BPW)],
                         idxn_v.at[pl.ds(j * BPW, BPW)], nsem[0])
        for j in range(S)
    ] + [pltpu.async_copy(nodes_hbm.at[pl.ds(base, BPW)], nodes_v, nsem[0])]
    for ld in idx_loads:
      ld.wait()
    _emit_pipeline(feat_hbm, self_out, sum_out, idxn_v, nodes_v,
                   accs, sbufs, nsem, wsem, ssem, swsem,
                   base, G_MAIN, BPW // G_MAIN)

  @pl.when(wid == NFULL)
  def _tail():
    base = NFULL * BPW
    idx_loads = [
        pltpu.async_copy(neigh_hbm.at[pl.ds(j * B + base, BT)],
                         idxn_v.at[pl.ds(j * BPW, BT)], nsem[0])
        for j in range(S)
    ] + [pltpu.async_copy(nodes_hbm.at[pl.ds(base, BT)],
                          nodes_v.at[pl.ds(0, BT)], nsem[0])]
    for ld in idx_loads:
      ld.wait()
    _emit_pipeline(feat_hbm, self_out, sum_out, idxn_v, nodes_v,
                   accs, sbufs, nsem, wsem, ssem, swsem,
                   base, G_TAIL, BT // G_TAIL)


def _sc_gather_mean(nodes, neigh_t, feat_table):
  mesh = plsc.VectorSubcoreMesh(core_axis_name="c", subcore_axis_name="s",
                                num_cores=NC, num_subcores=NS)
  f32 = jnp.float32
  out_type = (jax.ShapeDtypeStruct((B, F), f32),
              jax.ShapeDtypeStruct((B, F), f32))
  scratch = [
      pltpu.VMEM((S * BPW,), jnp.int32),
      pltpu.VMEM((BPW,), jnp.int32),
  ] + [pltpu.VMEM((G_MAIN, F), f32)] * (NACC + NSB) \
    + [pltpu.SemaphoreType.DMA] * (2 * NACC + 2 * NSB)
  return pl.kernel(_sc_body, out_type=out_type, mesh=mesh,
                   scratch_types=scratch)(nodes, neigh_t, feat_table)


def _tc_body(x1_ref, x2_ref, w1_ref, w2_ref, b_ref, o_ref):
  y = jnp.dot(x1_ref[...], w1_ref[...], preferred_element_type=jnp.float32)
  y = y + jnp.dot(x2_ref[...], w2_ref[...], preferred_element_type=jnp.float32)
  y = y + b_ref[...]
  o_ref[...] = y * jax.nn.sigmoid(y)


def _tc_linear_swish(x1, x2, w1, w2, b2d, bt=10000):
  nblk = B // bt
  return pl.pallas_call(
      _tc_body,
      grid=(nblk,),
      in_specs=[
          pl.BlockSpec((bt, F), lambda i: (i, 0)),
          pl.BlockSpec((bt, F), lambda i: (i, 0)),
          pl.BlockSpec((F, E), lambda i: (0, 0)),
          pl.BlockSpec((F, E), lambda i: (0, 0)),
          pl.BlockSpec((1, E), lambda i: (0, 0)),
      ],
      out_specs=pl.BlockSpec((bt, E), lambda i: (i, 0)),
      out_shape=jax.ShapeDtypeStruct((B, E), jnp.float32),
  )(x1, x2, w1, w2, b2d)


def kernel(nodes, neigh_idx, feat_table, W, b):
  self_f, sum_f = _sc_gather_mean(nodes, neigh_idx.T.reshape(-1), feat_table)
  w2s = W[F:] * jnp.float32(1.0 / S)
  return _tc_linear_swish(self_f, sum_f, W[:F], w2s, b.reshape(1, E))


# speedup vs baseline: 1.0264x; 1.0264x over previous
import jax
import jax.numpy as jnp
from jax import lax
from jax.experimental import pallas as pl
from jax.experimental.pallas import tpu as pltpu
from jax.experimental.pallas import tpu_sc as plsc

B = 30000
S = 16
F = 128
E = 64
NC = 2
NS = 16
NW = NC * NS
BPW = 960
NFULL = B // BPW
BT = B - NFULL * BPW
G_MAIN = 96
G_TAIL = 48
NACC = 4
NSB = 2


def _emit_pipeline(feat_hbm, self_out, sum_out, idxn_v, nodes_v,
                   accs, sbufs, nsem, wsem, ssem, swsem, base, g, nchunk):
  pend_init = {}
  pend_write = {}
  pend_sg = {}
  pend_sw = {}
  waited_write = set()
  waited_sw = set()

  def acc_ref(sl):
    return accs[sl].at[pl.ds(0, g)] if g != accs[sl].shape[0] else accs[sl]

  def sbuf_ref(sl):
    return sbufs[sl].at[pl.ds(0, g)] if g != sbufs[sl].shape[0] else sbufs[sl]

  for c in range(min(NACC, nchunk)):
    pend_init[c] = pltpu.async_copy(
        feat_hbm.at[idxn_v.at[pl.ds(c * g, g)]], acc_ref(c % NACC),
        nsem[c % NACC])
  for c in range(min(NSB, nchunk)):
    pend_sg[c] = pltpu.async_copy(
        feat_hbm.at[nodes_v.at[pl.ds(c * g, g)]], sbuf_ref(c % NSB),
        ssem[c % NSB])

  def fire_adds(c):
    return [
        pltpu.async_copy(feat_hbm.at[idxn_v.at[pl.ds(j * BPW + c * g, g)]],
                         acc_ref(c % NACC), nsem[c % NACC], add=True)
        for j in range(1, S)
    ]

  pend_init[0].wait()
  fired = {0: fire_adds(0)}

  for c in range(nchunk):
    sl = c % NACC
    ssl = c % NSB
    if c + 1 in pend_init:
      pend_init[c + 1].wait()
      fired[c + 1] = fire_adds(c + 1)
    adds = fired.pop(c)
    pend_sg[c].wait()
    pend_sw[c] = pltpu.async_copy(
        sbuf_ref(ssl), self_out.at[pl.ds(base + c * g, g)], swsem[ssl])
    if c + NSB < nchunk:
      pend_sw[c].wait()
      waited_sw.add(c)
      pend_sg[c + NSB] = pltpu.async_copy(
          feat_hbm.at[nodes_v.at[pl.ds((c + NSB) * g, g)]],
          sbuf_ref((c + NSB) % NSB), ssem[(c + NSB) % NSB])
    nxt = c + NACC - 1
    if c >= 1 and nxt < nchunk:
      pend_write[c - 1].wait()
      waited_write.add(c - 1)
      pend_init[nxt] = pltpu.async_copy(
          feat_hbm.at[idxn_v.at[pl.ds(nxt * g, g)]], acc_ref(nxt % NACC),
          nsem[nxt % NACC])
    for a in adds:
      a.wait()
    pend_write[c] = pltpu.async_copy(
        acc_ref(sl), sum_out.at[pl.ds(base + c * g, g)], wsem[sl])

  for c in range(nchunk):
    if c in pend_write and c not in waited_write:
      pend_write[c].wait()
    if c in pend_sw and c not in waited_sw:
      pend_sw[c].wait()


def _sc_body(nodes_hbm, neigh_hbm, feat_hbm, self_out, sum_out,
             idxn_v, nodes_v, acc0, acc1, acc2, acc3, sb0, sb1,
             nsem0, nsem1, nsem2, nsem3, wsem0, wsem1, wsem2, wsem3,
             ssem0, ssem1, swsem0, swsem1):
  accs = [acc0, acc1, acc2, acc3]
  nsem = [nsem0, nsem1, nsem2, nsem3]
  wsem = [wsem0, wsem1, wsem2, wsem3]
  sbufs = [sb0, sb1]
  ssem = [ssem0, ssem1]
  swsem = [swsem0, swsem1]

  c_ax = lax.axis_index("c")
  s_ax = lax.axis_index("s")
  wid = s_ax * NC + c_ax
  @pl.when(wid < NFULL)
  def _full():
    base = wid * BPW
    idx_loads = [
        pltpu.async_copy(neigh_hbm.at[pl.ds(j * B + base, BPW)],
                         idxn_v.at[pl.ds(j * BPW, BPW)], nsem[0])
        for j in range(S)
    ] + [pltpu.async_copy(nodes_hbm.at[pl.ds(base, BPW)], nodes_v, nsem[0])]
    for ld in idx_loads:
      ld.wait()
    _emit_pipeline(feat_hbm, self_out, sum_out, idxn_v, nodes_v,
                   accs, sbufs, nsem, wsem, ssem, swsem,
                   base, G_MAIN, BPW // G_MAIN)

  @pl.when(wid == NFULL)
  def _tail():
    base = NFULL * BPW
    idx_loads = [
        pltpu.async_copy(neigh_hbm.at[pl.ds(j * B + base, BT)],
                         idxn_v.at[pl.ds(j * BPW, BT)], nsem[0])
        for j in range(S)
    ] + [pltpu.async_copy(nodes_hbm.at[pl.ds(base, BT)],
                          nodes_v.at[pl.ds(0, BT)], nsem[0])]
    for ld in idx_loads:
      ld.wait()
    _emit_pipeline(feat_hbm, self_out, sum_out, idxn_v, nodes_v,
                   accs, sbufs, nsem, wsem, ssem, swsem,
                   base, G_TAIL, BT // G_TAIL)


def _sc_gather_mean(nodes, neigh_t, feat_table):
  mesh = plsc.VectorSubcoreMesh(core_axis_name="c", subcore_axis_name="s",
                                num_cores=NC, num_subcores=NS)
  f32 = jnp.float32
  out_type = (jax.ShapeDtypeStruct((B, F), f32),
              jax.ShapeDtypeStruct((B, F), f32))
  scratch = [
      pltpu.VMEM((S * BPW,), jnp.int32),
      pltpu.VMEM((BPW,), jnp.int32),
  ] + [pltpu.VMEM((G_MAIN, F), f32)] * (NACC + NSB) \
    + [pltpu.SemaphoreType.DMA] * (2 * NACC + 2 * NSB)
  return pl.kernel(_sc_body, out_type=out_type, mesh=mesh,
                   scratch_types=scratch)(nodes, neigh_t, feat_table)


def _tc_body(x1_ref, x2_ref, w1_ref, w2_ref, b_ref, o_ref):
  y = jnp.dot(x1_ref[...], w1_ref[...], preferred_element_type=jnp.float32)
  y = y + jnp.dot(x2_ref[...], w2_ref[...], preferred_element_type=jnp.float32)
  y = y + b_ref[...]
  o_ref[...] = y * jax.nn.sigmoid(y)


def _tc_linear_swish(x1, x2, w1, w2, b2d, bt=10000):
  nblk = B // bt
  return pl.pallas_call(
      _tc_body,
      grid=(nblk,),
      in_specs=[
          pl.BlockSpec((bt, F), lambda i: (i, 0)),
          pl.BlockSpec((bt, F), lambda i: (i, 0)),
          pl.BlockSpec((F, E), lambda i: (0, 0)),
          pl.BlockSpec((F, E), lambda i: (0, 0)),
          pl.BlockSpec((1, E), lambda i: (0, 0)),
      ],
      out_specs=pl.BlockSpec((bt, E), lambda i: (i, 0)),
      out_shape=jax.ShapeDtypeStruct((B, E), jnp.float32),
  )(x1, x2, w1, w2, b2d)


def kernel(nodes, neigh_idx, feat_table, W, b):
  self_f, sum_f = _sc_gather_mean(nodes, neigh_idx.T.reshape(-1), feat_table)
  w2s = W[F:] * jnp.float32(1.0 / S)
  return _tc_linear_swish(self_f, sum_f, W[:F], w2s, b.reshape(1, E))
